# baseline (speedup 1.0000x reference)
import functools

import jax
import jax.numpy as jnp
from jax import lax
from jax.experimental import pallas as pl
from jax.experimental.pallas import tpu as pltpu
from jax.experimental.pallas import tpu_sc as plsc

V = 1000000
D = 64
B = 16384
NS = 5

NC = 2
NSC = 16
NW = NC * NSC
EPW = B // NW
C = 128
NCHUNK = EPW // C
NR = C * NS

_mesh = plsc.VectorSubcoreMesh(
    core_axis_name="c", subcore_axis_name="s", num_cores=NC, num_subcores=NSC
)


@functools.partial(
    pl.kernel,
    out_type=jax.ShapeDtypeStruct((NW, 8, EPW), jnp.float32),
    mesh=_mesh,
    compiler_params=pltpu.CompilerParams(needs_layout_passes=False),
    scratch_types=[
        pltpu.VMEM((C,), jnp.int32),
        pltpu.VMEM((C,), jnp.int32),
        pltpu.VMEM((NR,), jnp.int32),
        pltpu.VMEM((C,), jnp.int32),
        pltpu.VMEM((C,), jnp.int32),
        pltpu.VMEM((NR,), jnp.int32),
        pltpu.VMEM((C, 128), jnp.float32),
        pltpu.VMEM((C, 128), jnp.float32),
        pltpu.VMEM((NR, 128), jnp.float32),
        pltpu.VMEM((8, C), jnp.float32),
        pltpu.SemaphoreType.DMA,
    ],
)
def _sc_dots(in_w, out_w, noise_w, in_tbl2, out_tbl2, dots_hbm,
             iidx, oidx, nidx, irow, orow, nrow, irows, orows, nrows,
             dots_v, sem):
    wid = lax.axis_index("s") * NC + lax.axis_index("c")
    lane = lax.iota(jnp.int32, 16)
    zeros16 = jnp.zeros((16,), jnp.float32)

    def chunk_body(c_i, carry):
        base = wid * EPW + c_i * C

        pltpu.sync_copy(in_w.at[pl.ds(base, C)], iidx)
        pltpu.sync_copy(out_w.at[pl.ds(base, C)], oidx)
        pltpu.sync_copy(noise_w.at[pl.ds(base * NS, NR)], nidx)

        for t in range(C // 16):
            s = pl.ds(t * 16, 16)
            irow[s] = lax.shift_right_logical(iidx[s], 1)
            orow[s] = lax.shift_right_logical(oidx[s], 1)
        for t in range(NR // 16):
            s = pl.ds(t * 16, 16)
            nrow[s] = lax.shift_right_logical(nidx[s], 1)

        cps = [
            pltpu.async_copy(in_tbl2.at[irow], irows, sem),
            pltpu.async_copy(out_tbl2.at[orow], orows, sem),
        ]
        for n in range(NS):
            cps.append(
                pltpu.async_copy(out_tbl2.at[nrow.at[pl.ds(n * C, C)]],
                                 nrows.at[pl.ds(n * C, C)], sem))
        for cp in cps:
            cp.wait()

        def group_body(g, carry2):
            e_vec = g * 16 + lane
            hin = (plsc.load_gather(iidx, [e_vec]) & 1) * 64
            hout = (plsc.load_gather(oidx, [e_vec]) & 1) * 64
            rvecs = []
            hns = []
            for n in range(NS):
                r_vec = e_vec * NS + n
                rvecs.append(r_vec)
                hns.append((plsc.load_gather(nidx, [r_vec]) & 1) * 64)
            accp = zeros16
            accn = [zeros16] * NS
            for d0 in range(D):
                dv = (d0 + lane) & 63
                a = plsc.load_gather(irows, [e_vec, hin + dv])
                b = plsc.load_gather(orows, [e_vec, hout + dv])
                accp = accp + a * b
                for n in range(NS):
                    cn = plsc.load_gather(nrows, [rvecs[n], hns[n] + dv])
                    accn[n] = accn[n] + a * cn
            s = pl.ds(g * 16, 16)
            dots_v[0, s] = accp
            for n in range(NS):
                dots_v[1 + n, s] = accn[n]
            dots_v[6, s] = zeros16
            dots_v[7, s] = zeros16
            return carry2

        lax.fori_loop(0, C // 16, group_body, 0)

        pltpu.sync_copy(dots_v, dots_hbm.at[wid, :, pl.ds(c_i * C, C)])
        return carry

    lax.fori_loop(0, NCHUNK, chunk_body, 0)


def _tc_loss_kernel(dots_ref, out_ref):
    x = dots_ref[...]
    row = lax.broadcasted_iota(jnp.int32, x.shape, 1)
    t = jnp.where(row == 0, x, -x)
    terms = jnp.log(1.0 / (1.0 + jnp.exp(-t)))
    terms = jnp.where(row < 6, terms, 0.0)
    out_ref[0, 0] = -jnp.sum(terms) / B


_tc_loss = pl.pallas_call(
    _tc_loss_kernel,
    out_shape=jax.ShapeDtypeStruct((1, 1), jnp.float32),
    out_specs=pl.BlockSpec(memory_space=pltpu.SMEM),
)


def kernel(input_words, output_words, noise_words, in_embed_weight, out_embed_weight):
    in_tbl2 = in_embed_weight.reshape(V // 2, 2 * D)
    out_tbl2 = out_embed_weight.reshape(V // 2, 2 * D)
    dots = _sc_dots(input_words, output_words, noise_words, in_tbl2, out_tbl2)
    return _tc_loss(dots)[0, 0]
